# Initial kernel scaffold; baseline (speedup 1.0000x reference)
import jax
import jax.numpy as jnp
from jax.experimental import pallas as pl


def kernel(xyz):
    raise NotImplementedError("write your pallas kernel here")


# speedup vs baseline: 1.2900x; 1.2900x over previous
import jax
import jax.numpy as jnp
from jax.experimental import pallas as pl
from jax.experimental.pallas import tpu as pltpu


def _fps_kernel(x_ref, out_ref):
    x = x_ref[0, 0]
    n = x.shape[1]
    iota = jax.lax.broadcasted_iota(jnp.int32, (1, n), 1)
    y = x[1:2, :]
    m0 = jnp.max(y)
    i0 = jnp.min(jnp.where(y == m0, iota, n))
    sel = iota == i0
    c = jnp.sum(jnp.where(sel, x, 0.0), axis=1, keepdims=True)
    d = x - c
    dist = jnp.sum(d * d, axis=0, keepdims=True)
    dist = jnp.minimum(dist, 1e10)
    m1 = jnp.max(dist)
    i1 = jnp.min(jnp.where(dist == m1, iota, n))
    out = jnp.concatenate(
        [jnp.full((1, 1), i0, jnp.int32), jnp.full((1, 1), i1, jnp.int32)], axis=1
    )
    out_ref[0] = out


def kernel(xyz):
    _, b, c, n = xyz.shape
    out = pl.pallas_call(
        _fps_kernel,
        grid=(b,),
        in_specs=[pl.BlockSpec((1, 1, c, n), lambda i: (0, i, 0, 0))],
        out_specs=pl.BlockSpec((1, 1, 2), lambda i: (i, 0, 0)),
        out_shape=jax.ShapeDtypeStruct((b, 1, 2), jnp.int32),
        compiler_params=pltpu.CompilerParams(
            dimension_semantics=("arbitrary",),
        ),
    )(xyz)
    return out.reshape(b, 2)
